# baseline (speedup 1.0000x reference)
import functools

import jax
import jax.numpy as jnp
from jax.experimental import pallas as pl
from jax.experimental.pallas import tpu as pltpu

SMOOTH = 0.1
PAD = 0
CONF = 1.0 - SMOOTH

BR = 256
BC = 2048


def _loss_kernel(x_ref, y_ref, out_ref, m_s, s_s, rs_s, xy_s, x0_s, *, c, ncb):
    i = pl.program_id(0)
    j = pl.program_id(1)

    xb = x_ref[...]

    @pl.when(j == 0)
    def _init():
        s_s[...] = jnp.zeros_like(s_s)
        rs_s[...] = jnp.zeros_like(rs_s)
        xy_s[...] = jnp.zeros_like(xy_s)
        m_s[...] = jnp.full_like(m_s, -jnp.inf)
        x0_s[...] = xb[:, 0:1]

    def update(xv, xz, col):
        bm = jnp.max(xv, axis=1, keepdims=True)
        new_m = jnp.maximum(m_s[...], bm)
        corr = jnp.exp(m_s[...] - new_m)
        eb = jnp.exp(xv - new_m)
        ones = jnp.ones((xv.shape[1], 1), jnp.float32)
        bs = jnp.dot(eb, ones, preferred_element_type=jnp.float32)
        s_s[...] = s_s[...] * corr + bs
        m_s[...] = new_m
        rs_s[...] = rs_s[...] + jnp.dot(
            xz, ones, preferred_element_type=jnp.float32)
        yv = y_ref[...]
        xy_s[...] = xy_s[...] + jnp.sum(
            jnp.where(col == yv, xz, 0.0), axis=1, keepdims=True)

    col = j * BC + jax.lax.broadcasted_iota(jnp.int32, xb.shape, 1)

    @pl.when(j < ncb - 1)
    def _full():
        update(xb, xb, col)

    @pl.when(j == ncb - 1)
    def _partial():
        inb = col < c
        update(jnp.where(inb, xb, -jnp.inf), jnp.where(inb, xb, 0.0), col)

        eps = SMOOTH / (c - 2)
        k_const = SMOOTH * jnp.log(jnp.float32(eps)) + CONF * jnp.log(
            jnp.float32(CONF))
        lse = m_s[...] + jnp.log(s_s[...])
        rest = rs_s[...] - x0_s[...] - xy_s[...] - (c - 2) * lse
        row = k_const - eps * rest - CONF * (xy_s[...] - lse)
        row = jnp.where(y_ref[...] != PAD, row, 0.0)
        part = jnp.sum(row, keepdims=True)

        @pl.when(i == 0)
        def _():
            out_ref[...] = part

        @pl.when(i != 0)
        def _():
            out_ref[...] = out_ref[...] + part


@jax.jit
def kernel(x, y):
    b, c = x.shape
    ncb = pl.cdiv(c, BC)
    y2 = y.astype(jnp.int32).reshape(b, 1)
    out = pl.pallas_call(
        functools.partial(_loss_kernel, c=c, ncb=ncb),
        grid=(b // BR, ncb),
        in_specs=[
            pl.BlockSpec((BR, BC), lambda i, j: (i, j)),
            pl.BlockSpec((BR, 1), lambda i, j: (i, 0)),
        ],
        out_specs=pl.BlockSpec((1, 1), lambda i, j: (0, 0)),
        out_shape=jax.ShapeDtypeStruct((1, 1), jnp.float32),
        scratch_shapes=[pltpu.VMEM((BR, 1), jnp.float32) for _ in range(5)],
    )(x, y2)
    return out[0, 0]


# speedup vs baseline: 1.0549x; 1.0549x over previous
import functools

import jax
import jax.numpy as jnp
from jax.experimental import pallas as pl
from jax.experimental.pallas import tpu as pltpu

SMOOTH = 0.1
PAD = 0
CONF = 1.0 - SMOOTH

BR = 256
BC = 2048


def _loss_kernel(x_ref, y_ref, out_ref, m_s, s_s, rs_s, xy_s, x0_s, *, c, ncb):
    j = pl.program_id(1)

    xb = x_ref[...]

    @pl.when(j == 0)
    def _init():
        s_s[...] = jnp.zeros_like(s_s)
        rs_s[...] = jnp.zeros_like(rs_s)
        xy_s[...] = jnp.zeros_like(xy_s)
        m_s[...] = jnp.full_like(m_s, -jnp.inf)
        x0_s[...] = xb[:, 0:1]

    def update(xv, xz, col):
        bm = jnp.max(xv, axis=1, keepdims=True)
        new_m = jnp.maximum(m_s[...], bm)
        corr = jnp.exp(m_s[...] - new_m)
        bs = jnp.sum(jnp.exp(xv - new_m), axis=1, keepdims=True)
        s_s[...] = s_s[...] * corr + bs
        m_s[...] = new_m
        rs_s[...] = rs_s[...] + jnp.sum(xz, axis=1, keepdims=True)
        yv = y_ref[...]
        xy_s[...] = xy_s[...] + jnp.sum(
            jnp.where(col == yv, xz, 0.0), axis=1, keepdims=True)

    col = j * BC + jax.lax.broadcasted_iota(jnp.int32, xb.shape, 1)

    @pl.when(j < ncb - 1)
    def _full():
        update(xb, xb, col)

    @pl.when(j == ncb - 1)
    def _partial():
        inb = col < c
        update(jnp.where(inb, xb, -jnp.inf), jnp.where(inb, xb, 0.0), col)

        eps = SMOOTH / (c - 2)
        k_const = SMOOTH * jnp.log(jnp.float32(eps)) + CONF * jnp.log(
            jnp.float32(CONF))
        lse = m_s[...] + jnp.log(s_s[...])
        rest = rs_s[...] - x0_s[...] - xy_s[...] - (c - 2) * lse
        row = k_const - eps * rest - CONF * (xy_s[...] - lse)
        row = jnp.where(y_ref[...] != PAD, row, 0.0)
        out_ref[...] = jnp.sum(row, keepdims=True)[None]


@jax.jit
def kernel(x, y):
    b, c = x.shape
    ncb = pl.cdiv(c, BC)
    nrb = b // BR
    y2 = y.astype(jnp.int32).reshape(b, 1)
    parts = pl.pallas_call(
        functools.partial(_loss_kernel, c=c, ncb=ncb),
        grid=(nrb, ncb),
        in_specs=[
            pl.BlockSpec((BR, BC), lambda i, j: (i, j)),
            pl.BlockSpec((BR, 1), lambda i, j: (i, 0)),
        ],
        out_specs=pl.BlockSpec((1, 1, 1), lambda i, j: (i, 0, 0)),
        out_shape=jax.ShapeDtypeStruct((nrb, 1, 1), jnp.float32),
        scratch_shapes=[pltpu.VMEM((BR, 1), jnp.float32) for _ in range(5)],
        compiler_params=pltpu.CompilerParams(
            dimension_semantics=("parallel", "arbitrary")),
    )(x, y2)
    return jnp.sum(parts)


# speedup vs baseline: 1.1488x; 1.0890x over previous
import functools

import jax
import jax.numpy as jnp
from jax.experimental import pallas as pl
from jax.experimental.pallas import tpu as pltpu

SMOOTH = 0.1
PAD = 0
CONF = 1.0 - SMOOTH

BR = 256
BC = 4096


def _loss_kernel(x_ref, y_ref, out_ref, m_s, s_s, rs_s, xy_s, x0_s, *, c, ncb):
    j = pl.program_id(1)

    xb = x_ref[...]

    @pl.when(j == 0)
    def _init():
        s_s[...] = jnp.zeros_like(s_s)
        rs_s[...] = jnp.zeros_like(rs_s)
        xy_s[...] = jnp.zeros_like(xy_s)
        m_s[...] = jnp.full_like(m_s, -jnp.inf)
        x0_s[...] = xb[:, 0:1]

    def update(xv, xz, col):
        bm = jnp.max(xv, axis=1, keepdims=True)
        new_m = jnp.maximum(m_s[...], bm)
        corr = jnp.exp(m_s[...] - new_m)
        bs = jnp.sum(jnp.exp(xv - new_m), axis=1, keepdims=True)
        s_s[...] = s_s[...] * corr + bs
        m_s[...] = new_m
        rs_s[...] = rs_s[...] + jnp.sum(xz, axis=1, keepdims=True)
        yv = y_ref[...]
        xy_s[...] = xy_s[...] + jnp.sum(
            jnp.where(col == yv, xz, 0.0), axis=1, keepdims=True)

    col = j * BC + jax.lax.broadcasted_iota(jnp.int32, xb.shape, 1)

    @pl.when(j < ncb - 1)
    def _full():
        update(xb, xb, col)

    @pl.when(j == ncb - 1)
    def _partial():
        inb = col < c
        update(jnp.where(inb, xb, -jnp.inf), jnp.where(inb, xb, 0.0), col)

        eps = SMOOTH / (c - 2)
        k_const = SMOOTH * jnp.log(jnp.float32(eps)) + CONF * jnp.log(
            jnp.float32(CONF))
        lse = m_s[...] + jnp.log(s_s[...])
        rest = rs_s[...] - x0_s[...] - xy_s[...] - (c - 2) * lse
        row = k_const - eps * rest - CONF * (xy_s[...] - lse)
        row = jnp.where(y_ref[...] != PAD, row, 0.0)
        out_ref[...] = jnp.sum(row, keepdims=True)[None]


@jax.jit
def kernel(x, y):
    b, c = x.shape
    ncb = pl.cdiv(c, BC)
    nrb = b // BR
    y2 = y.astype(jnp.int32).reshape(b, 1)
    parts = pl.pallas_call(
        functools.partial(_loss_kernel, c=c, ncb=ncb),
        grid=(nrb, ncb),
        in_specs=[
            pl.BlockSpec((BR, BC), lambda i, j: (i, j)),
            pl.BlockSpec((BR, 1), lambda i, j: (i, 0)),
        ],
        out_specs=pl.BlockSpec((1, 1, 1), lambda i, j: (i, 0, 0)),
        out_shape=jax.ShapeDtypeStruct((nrb, 1, 1), jnp.float32),
        scratch_shapes=[pltpu.VMEM((BR, 1), jnp.float32) for _ in range(5)],
        compiler_params=pltpu.CompilerParams(
            dimension_semantics=("parallel", "arbitrary")),
    )(x, y2)
    return jnp.sum(parts)


# speedup vs baseline: 1.2010x; 1.0455x over previous
import functools

import jax
import jax.numpy as jnp
from jax.experimental import pallas as pl
from jax.experimental.pallas import tpu as pltpu

SMOOTH = 0.1
PAD = 0
CONF = 1.0 - SMOOTH

BR = 256
BC = 8192


def _loss_kernel(x_ref, y_ref, out_ref, m_s, s_s, rs_s, xy_s, x0_s, *, c, ncb):
    j = pl.program_id(1)

    xb = x_ref[...]

    @pl.when(j == 0)
    def _init():
        s_s[...] = jnp.zeros_like(s_s)
        rs_s[...] = jnp.zeros_like(rs_s)
        xy_s[...] = jnp.zeros_like(xy_s)
        m_s[...] = jnp.full_like(m_s, -jnp.inf)
        x0_s[...] = xb[:, 0:1]

    def update(xv, xz, col):
        bm = jnp.max(xv, axis=1, keepdims=True)
        new_m = jnp.maximum(m_s[...], bm)
        corr = jnp.exp(m_s[...] - new_m)
        bs = jnp.sum(jnp.exp(xv - new_m), axis=1, keepdims=True)
        s_s[...] = s_s[...] * corr + bs
        m_s[...] = new_m
        rs_s[...] = rs_s[...] + jnp.sum(xz, axis=1, keepdims=True)
        yv = y_ref[...]
        xy_s[...] = xy_s[...] + jnp.sum(
            jnp.where(col == yv, xz, 0.0), axis=1, keepdims=True)

    col = j * BC + jax.lax.broadcasted_iota(jnp.int32, xb.shape, 1)

    @pl.when(j < ncb - 1)
    def _full():
        update(xb, xb, col)

    @pl.when(j == ncb - 1)
    def _partial():
        inb = col < c
        update(jnp.where(inb, xb, -jnp.inf), jnp.where(inb, xb, 0.0), col)

        eps = SMOOTH / (c - 2)
        k_const = SMOOTH * jnp.log(jnp.float32(eps)) + CONF * jnp.log(
            jnp.float32(CONF))
        lse = m_s[...] + jnp.log(s_s[...])
        rest = rs_s[...] - x0_s[...] - xy_s[...] - (c - 2) * lse
        row = k_const - eps * rest - CONF * (xy_s[...] - lse)
        row = jnp.where(y_ref[...] != PAD, row, 0.0)
        out_ref[...] = jnp.sum(row, keepdims=True)[None]


@jax.jit
def kernel(x, y):
    b, c = x.shape
    ncb = pl.cdiv(c, BC)
    nrb = b // BR
    y2 = y.astype(jnp.int32).reshape(b, 1)
    parts = pl.pallas_call(
        functools.partial(_loss_kernel, c=c, ncb=ncb),
        grid=(nrb, ncb),
        in_specs=[
            pl.BlockSpec((BR, BC), lambda i, j: (i, j)),
            pl.BlockSpec((BR, 1), lambda i, j: (i, 0)),
        ],
        out_specs=pl.BlockSpec((1, 1, 1), lambda i, j: (i, 0, 0)),
        out_shape=jax.ShapeDtypeStruct((nrb, 1, 1), jnp.float32),
        scratch_shapes=[pltpu.VMEM((BR, 1), jnp.float32) for _ in range(5)],
        compiler_params=pltpu.CompilerParams(
            dimension_semantics=("parallel", "arbitrary")),
    )(x, y2)
    return jnp.sum(parts)
